# speedup vs baseline: 1.7004x; 1.7004x over previous
import numpy as np
import jax
import jax.numpy as jnp
from jax.experimental import pallas as pl

HEAD = 16
ROWS = 16 * 72
OFFS = [(kh - 1) * 8 + (kw - 1) for kh in range(3) for kw in range(3)]
IBIG = np.int32(1 << 30)

_mask_np = np.zeros((9, 8), np.float32)
_mask_np[1:8, 1:8] = 1.0
_MASK = np.tile(_mask_np.reshape(-1), 16).reshape(ROWS, 1)


def _taps(W):
    return jnp.transpose(W, (2, 3, 1, 0)).reshape(9, W.shape[1], W.shape[0])


def _net(xp_ref, mask_ref,
         wt1_ref, b1_ref, wt2_ref, b2_ref, wtr1_ref, br1_ref, wr2_ref, br2_ref,
         embT_ref, ehi_ref, emid_ref, elo_ref,
         wtd1_ref, bd1_ref, wtr1d_ref, br1d_ref, wr2d_ref, br2d_ref,
         wtdt_ref, bdt_ref,
         xhat_ref, ze_ref, zq_ref):
    mask = mask_ref[...]

    def conv3(ap, wt_ref, b):
        acc = jax.lax.dot(ap[HEAD + OFFS[0]:HEAD + OFFS[0] + ROWS, :],
                          wt_ref[0], preferred_element_type=jnp.float32)
        for t in range(1, 9):
            acc = acc + jax.lax.dot(ap[HEAD + OFFS[t]:HEAD + OFFS[t] + ROWS, :],
                                    wt_ref[t], preferred_element_type=jnp.float32)
        return (acc + b) * mask

    def repad(y):
        z = jnp.zeros((HEAD, y.shape[1]), jnp.float32)
        return jnp.concatenate([z, y, z], axis=0)

    z1 = conv3(xp_ref[...], wt1_ref, b1_ref[...])
    z2 = conv3(repad(jax.nn.relu(z1)), wt2_ref, b2_ref[...])
    t = conv3(repad(jax.nn.relu(z2)), wtr1_ref, br1_ref[...])
    t = (jax.lax.dot(jax.nn.relu(t), wr2_ref[...],
                     preferred_element_type=jnp.float32) + br2_ref[...]) * mask
    z = z2 + t
    ze_ref[...] = z

    embT = embT_ref[...]
    e2 = jnp.sum(embT * embT, axis=0, keepdims=True)
    s = e2 - 2.0 * jax.lax.dot(z, embT, preferred_element_type=jnp.float32)
    kio = jax.lax.broadcasted_iota(jnp.int32, (ROWS, 512), 1)
    INF = jnp.float32(np.inf)

    def take_min(sc):
        m = jnp.min(sc, axis=1, keepdims=True)
        i = jnp.min(jnp.where(sc == m, kio, IBIG), axis=1, keepdims=True)
        return i, jnp.where(kio == i, INF, sc)

    i1, s = take_min(s)
    i2, s = take_min(s)
    i3, s = take_min(s)

    ehi, emid, elo = ehi_ref[...], emid_ref[...], elo_ref[...]

    def gather_exact(i):
        oh = (kio == i).astype(jnp.float32)
        g = (jax.lax.dot(oh, ehi, preferred_element_type=jnp.float32)
             + jax.lax.dot(oh, emid, preferred_element_type=jnp.float32))
        return g + jax.lax.dot(oh, elo, preferred_element_type=jnp.float32)

    def exact_dist(e):
        t = e - z
        t = t * t
        tiles = []
        for ti in range(2):
            base = 128 * ti
            acc = t[:, base:base + 8]
            for p in range(1, 16):
                acc = acc + t[:, base + 8 * p:base + 8 * p + 8]
            A = [acc[:, ss:ss + 1] for ss in range(8)]
            tiles.append(((A[3] + A[7]) + (A[1] + A[5]))
                         + ((A[2] + A[6]) + (A[0] + A[4])))
        return tiles[0] + tiles[1]

    e1c, e2c, e3c = gather_exact(i1), gather_exact(i2), gather_exact(i3)
    d1, d2, d3 = exact_dist(e1c), exact_dist(e2c), exact_dist(e3c)

    def better(da, ia, db, ib):
        take_b = (db < da) | ((db == da) & (ib < ia))
        return jnp.where(take_b, db, da), jnp.where(take_b, ib, ia)

    dw, iw = better(d1, i1, d2, i2)
    dw, iw = better(dw, iw, d3, i3)
    sel2 = (iw == i2).astype(jnp.float32)
    sel3 = (iw == i3).astype(jnp.float32)
    zq = e1c + sel2 * (e2c - e1c)
    zq = zq + sel3 * (e3c - zq)
    zq_ref[...] = zq

    zqm = zq * mask
    di = z + (zqm - z)
    h = conv3(repad(di), wtd1_ref, bd1_ref[...])
    t = conv3(repad(jax.nn.relu(h)), wtr1d_ref, br1d_ref[...])
    t = (jax.lax.dot(jax.nn.relu(t), wr2d_ref[...],
                     preferred_element_type=jnp.float32) + br2d_ref[...]) * mask
    h = h + t
    xhat_ref[...] = conv3(repad(h), wtdt_ref, bdt_ref[...])


def kernel(x, We1, be1, We2, be2, Wr1e, br1e, Wr2e, br2e, emb,
           Wd1, bd1, Wr1d, br1d, Wr2d, br2d, Wdt, bdt):
    x4 = x.reshape(16, 7, 7, 512)
    xpad = jnp.pad(x4, ((0, 0), (1, 1), (1, 0), (0, 0))).reshape(ROWS, 512)
    xpad = jnp.pad(xpad, ((HEAD, HEAD), (0, 0)))

    ehi = emb.astype(jnp.bfloat16).astype(jnp.float32)
    rem = emb - ehi
    emid = rem.astype(jnp.bfloat16).astype(jnp.float32)
    elo = rem - emid

    Wdt_c = jnp.transpose(jnp.flip(Wdt, (2, 3)), (1, 0, 2, 3))

    outs = pl.pallas_call(
        _net,
        out_shape=[
            jax.ShapeDtypeStruct((ROWS, 512), jnp.float32),
            jax.ShapeDtypeStruct((ROWS, 256), jnp.float32),
            jax.ShapeDtypeStruct((ROWS, 256), jnp.float32),
        ],
    )(
        xpad, jnp.asarray(_MASK),
        _taps(We1), be1.reshape(1, 256), _taps(We2), be2.reshape(1, 256),
        _taps(Wr1e), br1e.reshape(1, 256),
        jnp.transpose(Wr2e[:, :, 0, 0]), br2e.reshape(1, 256),
        jnp.transpose(emb), ehi, emid, elo,
        _taps(Wd1), bd1.reshape(1, 256), _taps(Wr1d), br1d.reshape(1, 256),
        jnp.transpose(Wr2d[:, :, 0, 0]), br2d.reshape(1, 256),
        _taps(Wdt_c), bdt.reshape(1, 512),
    )
    xh, ze, zq = outs

    def unpack(y, C):
        return jnp.transpose(y.reshape(16, 9, 8, C)[:, 1:8, 1:8, :], (0, 3, 1, 2))

    return (unpack(xh, 512), unpack(ze, 256), unpack(zq, 256))


# speedup vs baseline: 2.0073x; 1.1805x over previous
import numpy as np
import jax
import jax.numpy as jnp
from jax.experimental import pallas as pl

HEAD = 16
ROWS = 16 * 72
OFFS = [(kh - 1) * 8 + (kw - 1) for kh in range(3) for kw in range(3)]
IBIG = np.int32(1 << 30)

_mask_np = np.zeros((9, 8), np.float32)
_mask_np[1:8, 1:8] = 1.0
_MASK = np.tile(_mask_np.reshape(-1), 16).reshape(ROWS, 1)


def _taps(W):
    return jnp.transpose(W, (2, 3, 1, 0)).reshape(9, W.shape[1], W.shape[0]).astype(jnp.bfloat16)


def _dot(a, b):
    return jax.lax.dot(a, b, preferred_element_type=jnp.float32)


def _net(xp_ref, mask_ref, maskT_ref,
         wt1_ref, b1_ref, wt2_ref, b2_ref, wtr1_ref, br1_ref, wr2_ref, br2_ref,
         emb_ref, e2_ref, embT_ref,
         wtd1_ref, bd1_ref, wtr1d_ref, br1d_ref, wr2d_ref, br2d_ref,
         wtdt_ref, bdt_ref,
         xhatT_ref, zeT_ref, zqT_ref):
    mask = mask_ref[...]
    bf = jnp.bfloat16

    def conv3(ap, wt_ref, b):
        acc = _dot(ap[HEAD + OFFS[0]:HEAD + OFFS[0] + ROWS, :], wt_ref[0])
        for t in range(1, 9):
            acc = acc + _dot(ap[HEAD + OFFS[t]:HEAD + OFFS[t] + ROWS, :], wt_ref[t])
        return (acc + b) * mask

    def repad(y):
        z = jnp.zeros((HEAD, y.shape[1]), y.dtype)
        return jnp.concatenate([z, y, z], axis=0)

    z1 = conv3(xp_ref[...], wt1_ref, b1_ref[...])
    z2 = conv3(repad(jax.nn.relu(z1).astype(bf)), wt2_ref, b2_ref[...])
    t = conv3(repad(jax.nn.relu(z2).astype(bf)), wtr1_ref, br1_ref[...])
    t = (_dot(jax.nn.relu(t).astype(bf), wr2_ref[...]) + br2_ref[...]) * mask
    z = z2 + t
    zT = jnp.transpose(z, (1, 0))
    zeT_ref[...] = zT

    s = e2_ref[...] - 2.0 * _dot(emb_ref[...], zT.astype(bf))
    kio = jax.lax.broadcasted_iota(jnp.int32, (512, ROWS), 0)
    INF = jnp.float32(np.inf)

    def take_min(sc):
        m = jnp.min(sc, axis=0, keepdims=True)
        i = jnp.min(jnp.where(sc == m, kio, IBIG), axis=0, keepdims=True)
        return i, jnp.where(kio == i, INF, sc)

    i1, s = take_min(s)
    i2, s = take_min(s)
    i3, s = take_min(s)

    embT = embT_ref[...]

    def gather_exact(i):
        oh = (kio == i).astype(jnp.float32)
        return jax.lax.dot(embT, oh, preferred_element_type=jnp.float32,
                           precision=jax.lax.Precision.HIGHEST)

    def exact_dist(eT):
        t = eT - zT
        t = t * t
        tiles = []
        for ti in range(2):
            base = 128 * ti
            acc = t[base:base + 8, :]
            for p in range(1, 16):
                acc = acc + t[base + 8 * p:base + 8 * p + 8, :]
            A = [acc[ss:ss + 1, :] for ss in range(8)]
            tiles.append(((A[3] + A[7]) + (A[1] + A[5]))
                         + ((A[2] + A[6]) + (A[0] + A[4])))
        return tiles[0] + tiles[1]

    e1c, e2c, e3c = gather_exact(i1), gather_exact(i2), gather_exact(i3)
    d1, d2, d3 = exact_dist(e1c), exact_dist(e2c), exact_dist(e3c)

    def better(da, ia, db, ib):
        take_b = (db < da) | ((db == da) & (ib < ia))
        return jnp.where(take_b, db, da), jnp.where(take_b, ib, ia)

    dw, iw = better(d1, i1, d2, i2)
    dw, iw = better(dw, iw, d3, i3)
    sel2 = (iw == i2).astype(jnp.float32)
    sel3 = (iw == i3).astype(jnp.float32)
    zqT = e1c + sel2 * (e2c - e1c)
    zqT = zqT + sel3 * (e3c - zqT)
    zqT_ref[...] = zqT

    diT = zT + (zqT * maskT_ref[...] - zT)
    di = jnp.transpose(diT, (1, 0))
    h = conv3(repad(di.astype(bf)), wtd1_ref, bd1_ref[...])
    t = conv3(repad(jax.nn.relu(h).astype(bf)), wtr1d_ref, br1d_ref[...])
    t = (_dot(jax.nn.relu(t).astype(bf), wr2d_ref[...]) + br2d_ref[...]) * mask
    h = h + t
    xhat = conv3(repad(h.astype(bf)), wtdt_ref, bdt_ref[...])
    xhatT_ref[...] = jnp.transpose(xhat, (1, 0))


def kernel(x, We1, be1, We2, be2, Wr1e, br1e, Wr2e, br2e, emb,
           Wd1, bd1, Wr1d, br1d, Wr2d, br2d, Wdt, bdt):
    x4 = x.astype(jnp.bfloat16).reshape(16, 7, 7, 512)
    xpad = jnp.pad(x4, ((0, 0), (1, 1), (1, 0), (0, 0))).reshape(ROWS, 512)
    xpad = jnp.pad(xpad, ((HEAD, HEAD), (0, 0)))

    e2 = jnp.sum(emb * emb, axis=1, keepdims=True)

    Wdt_c = jnp.transpose(jnp.flip(Wdt, (2, 3)), (1, 0, 2, 3))

    outs = pl.pallas_call(
        _net,
        out_shape=[
            jax.ShapeDtypeStruct((512, ROWS), jnp.float32),
            jax.ShapeDtypeStruct((256, ROWS), jnp.float32),
            jax.ShapeDtypeStruct((256, ROWS), jnp.float32),
        ],
    )(
        xpad, jnp.asarray(_MASK), jnp.asarray(_MASK.T),
        _taps(We1), be1.reshape(1, 256), _taps(We2), be2.reshape(1, 256),
        _taps(Wr1e), br1e.reshape(1, 256),
        jnp.transpose(Wr2e[:, :, 0, 0]).astype(jnp.bfloat16), br2e.reshape(1, 256),
        emb.astype(jnp.bfloat16), e2, jnp.transpose(emb),
        _taps(Wd1), bd1.reshape(1, 256), _taps(Wr1d), br1d.reshape(1, 256),
        jnp.transpose(Wr2d[:, :, 0, 0]).astype(jnp.bfloat16), br2d.reshape(1, 256),
        _taps(Wdt_c), bdt.reshape(1, 512),
    )
    xhT, zeT, zqT = outs

    def unpack(yT, C):
        return jnp.transpose(yT.reshape(C, 16, 9, 8)[:, :, 1:8, 1:8], (1, 0, 2, 3))

    return (unpack(xhT, 512), unpack(zeT, 256), unpack(zqT, 256))


# speedup vs baseline: 2.3492x; 1.1703x over previous
import numpy as np
import jax
import jax.numpy as jnp
from jax.experimental import pallas as pl

HEAD = 16
ROWS = 16 * 72
OFFS = [(kh - 1) * 8 + (kw - 1) for kh in range(3) for kw in range(3)]
IBIG = np.int32(1 << 30)

_mask_np = np.zeros((9, 8), np.float32)
_mask_np[1:8, 1:8] = 1.0
_MASK = np.tile(_mask_np.reshape(-1), 16).reshape(ROWS, 1)


def _taps_oi(W):
    return jnp.transpose(W, (2, 3, 0, 1)).reshape(9, W.shape[0], W.shape[1]).astype(jnp.bfloat16)


def _dot(a, b):
    return jax.lax.dot(a, b, preferred_element_type=jnp.float32)


def _dot_t(a, b):
    return jax.lax.dot_general(a, b, (((1,), (1,)), ((), ())),
                               preferred_element_type=jnp.float32)


def _net(xp_ref, mask_ref, maskT_ref,
         wt1_ref, b1_ref, wt2_ref, b2_ref, wtr1_ref, br1_ref, wr2_ref, br2_ref,
         emb_ref, e2_ref, embT_ref,
         wtd1_ref, bd1_ref, wtr1d_ref, br1d_ref, wr2d_ref, br2d_ref,
         wtdt_ref, bdt_ref,
         xhatT_ref, zeT_ref, zqT_ref):
    mask = mask_ref[...]
    bf = jnp.bfloat16

    def conv3(ap, wt_ref, b, dot=_dot_t):
        acc = dot(ap[HEAD + OFFS[0]:HEAD + OFFS[0] + ROWS, :], wt_ref[0])
        for t in range(1, 9):
            acc = acc + dot(ap[HEAD + OFFS[t]:HEAD + OFFS[t] + ROWS, :], wt_ref[t])
        return (acc + b) * mask

    def repad(y):
        z = jnp.zeros((HEAD, y.shape[1]), y.dtype)
        return jnp.concatenate([z, y, z], axis=0)

    z1 = conv3(xp_ref[...], wt1_ref, b1_ref[...])
    z2 = conv3(repad(jax.nn.relu(z1).astype(bf)), wt2_ref, b2_ref[...])
    t = conv3(repad(jax.nn.relu(z2).astype(bf)), wtr1_ref, br1_ref[...])
    t = (_dot_t(jax.nn.relu(t).astype(bf), wr2_ref[...]) + br2_ref[...]) * mask
    z = z2 + t
    zT = jnp.transpose(z, (1, 0))
    zeT_ref[...] = zT

    s = e2_ref[...] - 2.0 * _dot(emb_ref[...], zT.astype(bf))
    kio = jax.lax.broadcasted_iota(jnp.int32, (512, ROWS), 0)
    INF = jnp.float32(np.inf)

    def take_min(sc):
        m = jnp.min(sc, axis=0, keepdims=True)
        i = jnp.min(jnp.where(sc == m, kio, IBIG), axis=0, keepdims=True)
        return i, jnp.where(kio == i, INF, sc)

    i1, s = take_min(s)
    i2, s = take_min(s)
    i3, s = take_min(s)

    embT = embT_ref[...]

    def gather_exact(i):
        oh = (kio == i).astype(jnp.float32)
        return jax.lax.dot(embT, oh, preferred_element_type=jnp.float32,
                           precision=jax.lax.Precision.HIGHEST)

    def exact_dist(eT):
        t = eT - zT
        t = t * t
        tiles = []
        for ti in range(2):
            base = 128 * ti
            acc = t[base:base + 8, :]
            for p in range(1, 16):
                acc = acc + t[base + 8 * p:base + 8 * p + 8, :]
            A = [acc[ss:ss + 1, :] for ss in range(8)]
            tiles.append(((A[3] + A[7]) + (A[1] + A[5]))
                         + ((A[2] + A[6]) + (A[0] + A[4])))
        return tiles[0] + tiles[1]

    e1c, e2c, e3c = gather_exact(i1), gather_exact(i2), gather_exact(i3)
    d1, d2, d3 = exact_dist(e1c), exact_dist(e2c), exact_dist(e3c)

    def better(da, ia, db, ib):
        take_b = (db < da) | ((db == da) & (ib < ia))
        return jnp.where(take_b, db, da), jnp.where(take_b, ib, ia)

    dw, iw = better(d1, i1, d2, i2)
    dw, iw = better(dw, iw, d3, i3)
    sel2 = (iw == i2).astype(jnp.float32)
    sel3 = (iw == i3).astype(jnp.float32)
    zqT = e1c + sel2 * (e2c - e1c)
    zqT = zqT + sel3 * (e3c - zqT)
    zqT_ref[...] = zqT

    diT = zT + (zqT * maskT_ref[...] - zT)
    di = jnp.transpose(diT, (1, 0))
    h = conv3(repad(di.astype(bf)), wtd1_ref, bd1_ref[...])
    t = conv3(repad(jax.nn.relu(h).astype(bf)), wtr1d_ref, br1d_ref[...])
    t = (_dot_t(jax.nn.relu(t).astype(bf), wr2d_ref[...]) + br2d_ref[...]) * mask
    h = h + t
    xhat = conv3(repad(h.astype(bf)), wtdt_ref, bdt_ref[...], dot=_dot)
    xhatT_ref[...] = jnp.transpose(xhat, (1, 0))


def kernel(x, We1, be1, We2, be2, Wr1e, br1e, Wr2e, br2e, emb,
           Wd1, bd1, Wr1d, br1d, Wr2d, br2d, Wdt, bdt):
    x4 = x.astype(jnp.bfloat16).reshape(16, 7, 7, 512)
    xpad = jnp.pad(x4, ((0, 0), (1, 1), (1, 0), (0, 0))).reshape(ROWS, 512)
    xpad = jnp.pad(xpad, ((HEAD, HEAD), (0, 0)))

    e2 = jnp.sum(emb * emb, axis=1, keepdims=True)

    wtdt = jnp.transpose(jnp.flip(Wdt, (2, 3)), (2, 3, 0, 1)).reshape(9, 256, 512).astype(jnp.bfloat16)

    outs = pl.pallas_call(
        _net,
        out_shape=[
            jax.ShapeDtypeStruct((512, ROWS), jnp.float32),
            jax.ShapeDtypeStruct((256, ROWS), jnp.float32),
            jax.ShapeDtypeStruct((256, ROWS), jnp.float32),
        ],
    )(
        xpad, jnp.asarray(_MASK), jnp.asarray(_MASK.T),
        _taps_oi(We1), be1.reshape(1, 256), _taps_oi(We2), be2.reshape(1, 256),
        _taps_oi(Wr1e), br1e.reshape(1, 256),
        Wr2e[:, :, 0, 0].astype(jnp.bfloat16), br2e.reshape(1, 256),
        emb.astype(jnp.bfloat16), e2, jnp.transpose(emb),
        _taps_oi(Wd1), bd1.reshape(1, 256), _taps_oi(Wr1d), br1d.reshape(1, 256),
        Wr2d[:, :, 0, 0].astype(jnp.bfloat16), br2d.reshape(1, 256),
        wtdt, bdt.reshape(1, 512),
    )
    xhT, zeT, zqT = outs

    def unpack(yT, C):
        return jnp.transpose(yT.reshape(C, 16, 9, 8)[:, :, 1:8, 1:8], (1, 0, 2, 3))

    return (unpack(xhT, 512), unpack(zeT, 256), unpack(zqT, 256))
